# baseline (speedup 1.0000x reference)
import functools

import jax
import jax.numpy as jnp
from jax import lax
from jax.experimental import pallas as pl
from jax.experimental.pallas import tpu as pltpu
from jax.experimental.pallas import tpu_sc as plsc

D = 128
V0, V1, V2 = 5, 6, 2
NC, NS = 2, 16
NW = NC * NS
SUB = 32


def _make_prep_body(lo, n_hi):
    def _prep_body(c0_ref, c1_ref, c2_ref, w0_ref, w1_ref, w2_ref,
                   codes_ref, t_ref):
        i0 = jnp.clip(c0_ref[...], 0, V0 - 1)
        i1 = jnp.clip(c1_ref[...], 0, V1 - 1)
        i2 = jnp.clip(c2_ref[...], 0, V2 - 1)
        code = i0 * (V1 * V2) + i1 * V2 + i2
        g = lax.broadcasted_iota(jnp.int32, code.shape, 0)
        c = g // 2
        owner = jnp.where(c < (lo + 1) * n_hi,
                          c // (lo + 1), (c - n_hi) // lo)
        lane = lax.broadcasted_iota(jnp.int32, code.shape, 1)
        rep = owner * SUB + (lane % SUB)
        codes_ref[...] = code + rep * (V0 * V1 * V2)

        w0 = w0_ref[...] * (1.0 / 3.0)
        w1 = w1_ref[...] * (1.0 / 3.0)
        w2 = w2_ref[...] * (1.0 / 3.0)
        rows = []
        for a in range(V0):
            for b in range(V1):
                rows.append((w0[a:a + 1, :] + w1[b:b + 1, :]) + w2)
        t = jnp.concatenate(rows, axis=0)
        for w in range(NW * SUB):
            t_ref[pl.ds(w * (V0 * V1 * V2), V0 * V1 * V2), :] = t

    return _prep_body


def _make_sc_gather(n_groups, lo, n_hi):
    mesh = plsc.VectorSubcoreMesh(core_axis_name="c", subcore_axis_name="s")
    CH = 2
    NSLOT = 3
    n_full = lo + (1 if n_hi else 0)
    n_bodies = (n_full + NSLOT - 1) // NSLOT

    @functools.partial(
        pl.kernel,
        out_type=jax.ShapeDtypeStruct((n_groups, D, D), jnp.float32),
        mesh=mesh,
        scratch_types=[
            pltpu.VMEM((n_full, CH, D), jnp.int32),
            pltpu.VMEM((NSLOT, CH, D, D), jnp.float32),
            pltpu.SemaphoreType.DMA,
            pltpu.SemaphoreType.DMA,
            pltpu.SemaphoreType.DMA,
            pltpu.SemaphoreType.DMA,
            pltpu.SemaphoreType.DMA,
            pltpu.SemaphoreType.DMA,
        ],
    )
    def sc_gather(t_hbm, codes_hbm, out_hbm, idx_v, rows_v,
                  g0, g1, g2, o0, o1, o2):
        wid = lax.axis_index("s") * NC + lax.axis_index("c")
        base_c = wid * lo + jnp.minimum(wid, n_hi)
        n_w = lo + (wid < n_hi).astype(jnp.int32)
        gsems = (g0, g1, g2)
        osems = (o0, o1, o2)

        pltpu.sync_copy(codes_hbm.at[pl.ds(base_c, lo)],
                        idx_v.at[pl.ds(0, lo)])
        if n_hi:
            @pl.when(wid < n_hi)
            def _():
                pltpu.sync_copy(codes_hbm.at[pl.ds(base_c + lo, 1)],
                                idx_v.at[pl.ds(lo, 1)])

        def fire(k, s):
            for j in range(CH):
                pltpu.async_copy(t_hbm.at[idx_v.at[k, j]],
                                 rows_v.at[s, j], gsems[s])

        def drain_g(s):
            pltpu.make_async_copy(out_hbm.at[pl.ds(0, CH)],
                                  rows_v.at[s], gsems[s]).wait()

        def fire_out(k, s):
            pltpu.async_copy(rows_v.at[s],
                             out_hbm.at[pl.ds((base_c + k) * CH, CH)],
                             osems[s])

        def drain_out(s):
            pltpu.make_async_copy(rows_v.at[s],
                                  out_hbm.at[pl.ds(0, CH)], osems[s]).wait()

        fire(0, 0)
        fire(1, 1)

        def body(m, carry):
            for t in range(NSLOT):
                k = m * NSLOT + t

                @pl.when(k < n_w)
                def _():
                    drain_g(t)
                    fire_out(k, t)
                s2 = (t + 2) % NSLOT

                @pl.when(jnp.logical_and(k >= 1, k + 2 < n_w))
                def _():
                    drain_out(s2)

                @pl.when(k + 2 < n_w)
                def _():
                    fire(k + 2, s2)

            return carry

        lax.fori_loop(0, n_bodies, body, 0)
        for s in range(NSLOT):
            drain_out(s)

    return sc_gather


def kernel(edge_attr, W0, W1, W2):
    E = edge_attr.shape[0]
    n_groups = E // D
    assert n_groups * D == E

    ea = edge_attr.astype(jnp.int32)
    c0 = ea[:, 0].reshape(n_groups, D)
    c1 = ea[:, 1].reshape(n_groups, D)
    c2 = ea[:, 2].reshape(n_groups, D)

    n_chunks = n_groups // 2
    lo = n_chunks // NW
    n_hi = n_chunks - lo * NW
    codes2d, table = pl.pallas_call(
        _make_prep_body(lo, n_hi),
        out_shape=[
            jax.ShapeDtypeStruct((n_groups, D), jnp.int32),
            jax.ShapeDtypeStruct((NW * SUB * V0 * V1 * V2, D), jnp.float32),
        ],
    )(c0, c1, c2, W0, W1, W2)

    codes3d = codes2d.reshape(n_chunks, 2, D)
    out3d = _make_sc_gather(n_groups, lo, n_hi)(table, codes3d)
    return out3d.reshape(E, D)
